# baseline (device time: 14651 ns/iter reference)
import jax
import jax.numpy as jnp
from jax import lax
from jax.experimental import pallas as pl
from jax.experimental.pallas import tpu as pltpu

N_DEV = 4


def kernel(x, router_W, route_idx, expert_W, shared_W):
    n, d = x.shape
    n_exp = router_W.shape[1]
    e_per, _, h = expert_W.shape
    chunk = n // N_DEV

    def body(x_ref, rw_ref, idx_ref, ew_ref, sw_ref, out_ref,
             w_ref, send_ref, comm_ref, send_sems, recv_sems):
        my_pos = lax.axis_index("i")

        barrier_sem = pltpu.get_barrier_semaphore()
        for o in range(1, N_DEV):
            pl.semaphore_signal(
                barrier_sem, inc=1,
                device_id=(lax.rem(my_pos + o, N_DEV),),
                device_id_type=pl.DeviceIdType.MESH,
            )
        pl.semaphore_wait(barrier_sem, N_DEV - 1)

        xv = x_ref[:, :]
        scores = jnp.dot(xv, rw_ref[:, :], preferred_element_type=jnp.float32)
        s_max = jnp.max(scores, axis=-1, keepdims=True)
        e = jnp.exp(scores - s_max)
        probs = e / jnp.sum(e, axis=-1, keepdims=True)
        cols = lax.broadcasted_iota(jnp.int32, (n, n_exp), 1)
        idx = idx_ref[:, :]
        for e_local in range(e_per):
            ge = my_pos * e_per + e_local
            p_e = jnp.sum(probs * (cols == ge).astype(jnp.float32),
                          axis=-1, keepdims=True)
            w_ref[:, e_local:e_local + 1] = p_e * (idx == ge).astype(jnp.float32)

        ew0 = ew_ref[0].astype(jnp.bfloat16)
        ew1 = ew_ref[1].astype(jnp.bfloat16)

        def chunk_partial(start):
            xr = x_ref[pl.ds(start, chunk), :]
            wr = w_ref[pl.ds(start, chunk), :]
            xs0 = (xr * wr[:, 0:1]).astype(jnp.bfloat16)
            xs1 = (xr * wr[:, 1:2]).astype(jnp.bfloat16)
            return (jnp.dot(xs0, ew0, preferred_element_type=jnp.float32)
                    + jnp.dot(xs1, ew1, preferred_element_type=jnp.float32))

        rdmas = []
        for o in range(1, N_DEV):
            q = lax.rem(my_pos + o, N_DEV)
            send_ref[o - 1] = chunk_partial(q * chunk).astype(jnp.bfloat16)
            rdma = pltpu.make_async_remote_copy(
                src_ref=send_ref.at[o - 1],
                dst_ref=comm_ref.at[o - 1],
                send_sem=send_sems.at[o - 1],
                recv_sem=recv_sems.at[o - 1],
                device_id=(q,),
                device_id_type=pl.DeviceIdType.MESH,
            )
            rdma.start()
            rdmas.append(rdma)

        mine = chunk_partial(my_pos * chunk)
        x_mine = x_ref[pl.ds(my_pos * chunk, chunk), :]
        mine = mine + jnp.dot(x_mine.astype(jnp.bfloat16),
                              sw_ref[:, :].astype(jnp.bfloat16),
                              preferred_element_type=jnp.float32)

        for rdma in rdmas:
            rdma.wait_recv()
        out_ref[:, :] = (mine
                         + comm_ref[0].astype(jnp.float32)
                         + comm_ref[1].astype(jnp.float32)
                         + comm_ref[2].astype(jnp.float32))
        for rdma in rdmas:
            rdma.wait_send()

    return pl.pallas_call(
        body,
        out_shape=jax.ShapeDtypeStruct((chunk, h), jnp.float32),
        in_specs=[pl.BlockSpec(memory_space=pltpu.VMEM)] * 5,
        out_specs=pl.BlockSpec(memory_space=pltpu.VMEM),
        scratch_shapes=[
            pltpu.VMEM((n, e_per), jnp.float32),
            pltpu.VMEM((N_DEV - 1, chunk, h), jnp.bfloat16),
            pltpu.VMEM((N_DEV - 1, chunk, h), jnp.bfloat16),
            pltpu.SemaphoreType.DMA((N_DEV - 1,)),
            pltpu.SemaphoreType.DMA((N_DEV - 1,)),
        ],
        compiler_params=pltpu.CompilerParams(collective_id=0),
    )(x, router_W, route_idx, expert_W, shared_W)
